# baseline (speedup 1.0000x reference)
import functools

import jax
import jax.numpy as jnp
from jax import lax
from jax.experimental import pallas as pl
from jax.experimental.pallas import tpu as pltpu
from jax.experimental.pallas import tpu_sc as plsc

N_TABLES = 26
VOCAB = 100000
DIM = 64
BATCH = 4096
NFEAT = N_TABLES + 1
NPAIR = NFEAT * (NFEAT - 1) // 2
BBLK = 512
IDX_CHUNK = 128


def _sc_gather(table_flat, idx3d):
    info = plsc.get_sparse_core_info()
    nw = info.num_cores * info.num_subcores
    tot = N_TABLES * BATCH
    per_w = tot // nw
    n_chunks = per_w // IDX_CHUNK

    mesh = plsc.VectorSubcoreMesh(core_axis_name="c", subcore_axis_name="s")

    @functools.partial(
        pl.kernel,
        mesh=mesh,
        compiler_params=pltpu.CompilerParams(use_tc_tiling_on_sc=False),
        out_type=jax.ShapeDtypeStruct((tot, DIM), jnp.float32),
        scratch_types=[
            pltpu.VMEM((n_chunks, IDX_CHUNK), jnp.int32),
            pltpu.VMEM((2, IDX_CHUNK, DIM), jnp.float32),
            pltpu.SemaphoreType.DMA,
            pltpu.SemaphoreType.DMA,
        ],
    )
    def gather(table_hbm, idx_hbm, out_hbm, idx_v, rows_v, sem0, sem1):
        wid = lax.axis_index("s") * info.num_cores + lax.axis_index("c")
        base_row = wid * n_chunks
        pltpu.sync_copy(idx_hbm.at[wid], idx_v)
        sems = (sem0, sem1)

        def start(c):
            return pltpu.async_copy(
                table_hbm.at[idx_v.at[c]], rows_v.at[c % 2], sems[c % 2])

        h = start(0)
        for c in range(n_chunks):
            h_next = start(c + 1) if c + 1 < n_chunks else None
            h.wait()
            pltpu.sync_copy(
                rows_v.at[c % 2],
                out_hbm.at[pl.ds((base_row + c) * IDX_CHUNK, IDX_CHUNK)])
            h = h_next

    return gather(table_flat, idx3d)


def _tc_body(dxt_ref, ly_ref,
             bw0_ref, bb0_ref, bw1_ref, bb1_ref, bw2_ref, bb2_ref,
             tw0_ref, tb0_ref, tw1_ref, tb1_ref, tw2_ref, tb2_ref,
             out_ref, tt_ref, rt_ref):
    f32 = jnp.float32
    x = jnp.maximum(jnp.dot(bw0_ref[...], dxt_ref[...],
                            preferred_element_type=f32) + bb0_ref[...], 0.0)
    x = jnp.maximum(jnp.dot(bw1_ref[...], x,
                            preferred_element_type=f32) + bb1_ref[...], 0.0)
    x = jnp.maximum(jnp.dot(bw2_ref[...], x,
                            preferred_element_type=f32) + bb2_ref[...], 0.0)
    tt_ref[0:DIM, :] = x
    tt_ref[DIM:, :] = ly_ref[...].T
    rt_ref[0:DIM, :] = x

    def pair_body(p, _):
        pf = (8 * p + 1).astype(f32)
        i = jnp.floor((1.0 + jnp.sqrt(pf)) * 0.5).astype(jnp.int32)
        j = p - (i * (i - 1)) // 2
        a = tt_ref[pl.ds(i * DIM, DIM), :]
        b = tt_ref[pl.ds(j * DIM, DIM), :]
        rt_ref[pl.ds(DIM + p, 1), :] = jnp.sum(a * b, axis=0)[None, :]
        return 0

    lax.fori_loop(0, NPAIR, pair_body, 0, unroll=8)

    z = jnp.maximum(jnp.dot(tw0_ref[...], rt_ref[...],
                            preferred_element_type=f32) + tb0_ref[...], 0.0)
    z = jnp.maximum(jnp.dot(tw1_ref[...], z,
                            preferred_element_type=f32) + tb1_ref[...], 0.0)
    z = jnp.dot(tw2_ref[...], z, preferred_element_type=f32) + tb2_ref[...]
    out_ref[...] = 1.0 / (1.0 + jnp.exp(-z))


def _tc_forward(dxt, ly, bw0, bb0, bw1, bb1, bw2, bb2,
                tw0, tb0, tw1, tb1, tw2, tb2):
    n_blocks = BATCH // BBLK
    full = lambda shape: pl.BlockSpec(shape, lambda b: (0, 0))
    in_specs = [
            pl.BlockSpec((dxt.shape[0], BBLK), lambda b: (0, b)),
            pl.BlockSpec((BBLK, N_TABLES * DIM), lambda b: (b, 0)),
            full(bw0.shape), full(bb0.shape),
            full(bw1.shape), full(bb1.shape),
            full(bw2.shape), full(bb2.shape),
            full(tw0.shape), full(tb0.shape),
            full(tw1.shape), full(tb1.shape),
            full(tw2.shape), full(tb2.shape),
    ]
    return pl.pallas_call(
        _tc_body,
        grid=(n_blocks,),
        in_specs=in_specs,
        out_specs=pl.BlockSpec((1, BBLK), lambda b: (0, b)),
        out_shape=jax.ShapeDtypeStruct((1, BATCH), jnp.float32),
        scratch_shapes=[
            pltpu.VMEM((NFEAT * DIM, BBLK), jnp.float32),
            pltpu.VMEM((DIM + NPAIR, BBLK), jnp.float32),
        ],
    )(dxt, ly, bw0, bb0, bw1, bb1, bw2, bb2, tw0, tb0, tw1, tb1, tw2, tb2)


def kernel(dense_x, lS_i, emb_W,
           bot_W0, bot_b0, bot_W1, bot_b1, bot_W2, bot_b2,
           top_W0, top_b0, top_W1, top_b1, top_W2, top_b2):
    idx = (lS_i[:, :, 0].astype(jnp.int32)
           + (jnp.arange(N_TABLES, dtype=jnp.int32) * VOCAB)[:, None])
    idx3d = idx.T.reshape(32, -1, IDX_CHUNK)
    ly = _sc_gather(emb_W.reshape(N_TABLES * VOCAB, DIM), idx3d)
    ly = ly.reshape(BATCH, N_TABLES * DIM)

    dxt = jnp.pad(dense_x.T, ((0, 3), (0, 0)))
    bw0 = jnp.pad(bot_W0, ((0, 0), (0, 3)))
    col = lambda v: v[:, None]

    out = _tc_forward(dxt, ly,
                      bw0, col(bot_b0), bot_W1, col(bot_b1), bot_W2, col(bot_b2),
                      top_W0, col(top_b0), top_W1, col(top_b1),
                      top_W2, col(top_b2))
    return out.reshape(BATCH, 1)
